# Initial kernel scaffold; baseline (speedup 1.0000x reference)
import jax
import jax.numpy as jnp
from jax.experimental import pallas as pl


def kernel(cls_logits, bbox_regression, bbox_ctrness, anchors):
    raise NotImplementedError("write your pallas kernel here")


# speedup vs baseline: 6.4162x; 6.4162x over previous
import functools

import jax
import jax.numpy as jnp
from jax.experimental import pallas as pl
from jax.experimental.pallas import tpu as pltpu

_N = 20000
_ROWS, _LANES = 160, 128
_NPAD = _ROWS * _LANES
_NCLS = 80
_K = 1000
_SCORE_THRESH = 0.2
_NMS_THRESH = 0.6
_NUM_OUT = 100
_IMG = 1024.0
_SCORE_BLOCK = 2000


def _score_body(logits_ref, ctr_ref, score_ref, label_ref):
    x = logits_ref[...]
    m = jnp.max(x, axis=1, keepdims=True)
    lane = jax.lax.broadcasted_iota(jnp.int32, x.shape, 1)
    label_ref[...] = jnp.min(
        jnp.where(x == m, lane, _NCLS), axis=1, keepdims=True)
    s = jnp.sqrt(jax.nn.sigmoid(m) * jax.nn.sigmoid(ctr_ref[...]))
    score_ref[...] = jnp.where(s > _SCORE_THRESH, s, 0.0)


def _nms_body(score_ref, label_ref, rx1_ref, ry1_ref, rx2_ref, ry2_ref,
              ax1_ref, ay1_ref, ax2_ref, ay2_ref, out_ref,
              nx1_r, ny1_r, nx2_r, ny2_r, area_r, live_r,
              bx1_r, by1_r, bx2_r, by2_r, cs_r, lab_r):
    s = score_ref[...]
    si = jax.lax.bitcast_convert_type(s, jnp.int32)
    row = jax.lax.broadcasted_iota(jnp.int32, (_ROWS, _LANES), 0)
    col = jax.lax.broadcasted_iota(jnp.int32, (_ROWS, _LANES), 1)
    gidx = row * _LANES + col

    def count(mask):
        return jnp.sum(mask.astype(jnp.int32))

    t = jnp.int32(0)
    for b in range(30, -1, -1):
        tc = t | jnp.int32(1 << b)
        t = jnp.where(count(si >= tc) >= _K, tc, t)
    eqt = si == t
    need = _K - count(si > t)
    x_cut = jnp.int32(0)
    for b in range(14, -1, -1):
        xc = x_cut | jnp.int32(1 << b)
        x_cut = jnp.where(count(eqt & (gidx < xc)) < need, xc, x_cut)
    cand = (si > t) | (eqt & (gidx <= x_cut))

    cx = (ax1_ref[...] + ax2_ref[...]) / 2.0
    cy = (ay1_ref[...] + ay2_ref[...]) / 2.0
    bx1 = jnp.clip(cx - rx1_ref[...], 0.0, _IMG)
    by1 = jnp.clip(cy - ry1_ref[...], 0.0, _IMG)
    bx2 = jnp.clip(cx + rx2_ref[...], 0.0, _IMG)
    by2 = jnp.clip(cy + ry2_ref[...], 0.0, _IMG)
    labf = label_ref[...].astype(jnp.float32)
    m4 = jnp.maximum(jnp.maximum(bx1, by1), jnp.maximum(bx2, by2))
    mc = jnp.max(jnp.where(cand, m4, 0.0))
    off = labf * (mc + 1.0)
    nx1 = bx1 + off
    ny1 = by1 + off
    nx2 = bx2 + off
    ny2 = by2 + off
    nx1_r[...] = nx1
    ny1_r[...] = ny1
    nx2_r[...] = nx2
    ny2_r[...] = ny2
    area_r[...] = (nx2 - nx1) * (ny2 - ny1)
    live_r[...] = jnp.where(cand, s, -jnp.inf)
    bx1_r[...] = bx1
    by1_r[...] = by1
    bx2_r[...] = bx2
    by2_r[...] = by2
    cs_r[...] = jnp.where(cand, s, 0.0)
    lab_r[...] = labf

    lanevec = jax.lax.broadcasted_iota(jnp.int32, (1, _LANES), 1)

    def step(i, first):
        live = live_r[...]
        m = jnp.max(live)
        idx = jnp.min(jnp.where(live == m, gidx, _NPAD))
        idx = jnp.where(m == -jnp.inf, first, idx)
        first = jnp.where(i == 0, idx, first)
        r = idx // _LANES
        c = idx - r * _LANES

        def pick(ref):
            return jnp.sum(jnp.where(lanevec == c, ref[pl.ds(r, 1), :], 0.0))

        px1b = pick(bx1_r)
        py1b = pick(by1_r)
        px2b = pick(bx2_r)
        py2b = pick(by2_r)
        psc = pick(cs_r)
        plab = pick(lab_r)
        poff = plab * (mc + 1.0)
        px1 = px1b + poff
        py1 = py1b + poff
        px2 = px2b + poff
        py2 = py2b + poff
        parea = (px2 - px1) * (py2 - py1)
        ltx = jnp.maximum(px1, nx1_r[...])
        lty = jnp.maximum(py1, ny1_r[...])
        rbx = jnp.minimum(px2, nx2_r[...])
        rby = jnp.minimum(py2, ny2_r[...])
        w = jnp.maximum(rbx - ltx, 0.0)
        h = jnp.maximum(rby - lty, 0.0)
        inter = w * h
        iou = inter / (parea + area_r[...] - inter + 1e-9)
        live_r[...] = jnp.where((iou > _NMS_THRESH) | (gidx == idx),
                                -jnp.inf, live)
        rowout = jnp.where(lanevec == 0, px1b,
                 jnp.where(lanevec == 1, py1b,
                 jnp.where(lanevec == 2, px2b,
                 jnp.where(lanevec == 3, py2b,
                 jnp.where(lanevec == 4, psc,
                 jnp.where(lanevec == 5, plab, 0.0))))))
        out_ref[pl.ds(i, 1), :] = rowout
        return first

    jax.lax.fori_loop(0, _NUM_OUT, step, jnp.int32(0))


@jax.jit
def kernel(cls_logits, bbox_regression, bbox_ctrness, anchors):
    grid = _N // _SCORE_BLOCK
    scores, labels = pl.pallas_call(
        _score_body,
        grid=(grid,),
        in_specs=[
            pl.BlockSpec((_SCORE_BLOCK, _NCLS), lambda i: (i, 0)),
            pl.BlockSpec((_SCORE_BLOCK, 1), lambda i: (i, 0)),
        ],
        out_specs=[
            pl.BlockSpec((_SCORE_BLOCK, 1), lambda i: (i, 0)),
            pl.BlockSpec((_SCORE_BLOCK, 1), lambda i: (i, 0)),
        ],
        out_shape=[
            jax.ShapeDtypeStruct((_N, 1), jnp.float32),
            jax.ShapeDtypeStruct((_N, 1), jnp.int32),
        ],
    )(cls_logits, bbox_ctrness)

    def fold(v):
        return jnp.pad(v.reshape(_N), (0, _NPAD - _N)).reshape(_ROWS, _LANES)

    packed = [fold(scores), fold(labels)]
    packed += [fold(bbox_regression[:, j]) for j in range(4)]
    packed += [fold(anchors[:, j]) for j in range(4)]

    out = pl.pallas_call(
        _nms_body,
        in_specs=[pl.BlockSpec(memory_space=pltpu.VMEM)] * 10,
        out_specs=pl.BlockSpec(memory_space=pltpu.VMEM),
        out_shape=jax.ShapeDtypeStruct((104, _LANES), jnp.float32),
        scratch_shapes=[pltpu.VMEM((_ROWS, _LANES), jnp.float32)] * 12,
    )(*packed)
    dets = out[:_NUM_OUT, :5]
    labels_out = out[:_NUM_OUT, 5].astype(jnp.int32)
    return dets, labels_out


# speedup vs baseline: 6.4921x; 1.0118x over previous
import jax
import jax.numpy as jnp
from jax.experimental import pallas as pl
from jax.experimental.pallas import tpu as pltpu

_N = 20000
_ROWS, _LANES = 160, 128
_NPAD = _ROWS * _LANES
_NCLS = 80
_K = 1000
_SCORE_THRESH = 0.2
_NMS_THRESH = 0.6
_NUM_OUT = 100
_IMG = 1024.0
_BLK = 2000


def _score_body(logits_ref, ctr_ref, box_ref, anc_ref, out_ref):
    x = logits_ref[...]
    m = jnp.max(x, axis=1, keepdims=True)
    lane = jax.lax.broadcasted_iota(jnp.int32, x.shape, 1)
    lab = jnp.min(jnp.where(x == m, lane, _NCLS), axis=1, keepdims=True)
    s = jnp.sqrt(jax.nn.sigmoid(m) * jax.nn.sigmoid(ctr_ref[...]))
    s = jnp.where(s > _SCORE_THRESH, s, 0.0)
    a = anc_ref[...]
    b = box_ref[...]
    cx = (a[:, 0:1] + a[:, 2:3]) / 2.0
    cy = (a[:, 1:2] + a[:, 3:4]) / 2.0
    bx1 = jnp.clip(cx - b[:, 0:1], 0.0, _IMG)
    by1 = jnp.clip(cy - b[:, 1:2], 0.0, _IMG)
    bx2 = jnp.clip(cx + b[:, 2:3], 0.0, _IMG)
    by2 = jnp.clip(cy + b[:, 3:4], 0.0, _IMG)
    z = jnp.zeros_like(s)
    out_ref[...] = jnp.concatenate(
        [bx1, by1, bx2, by2, s, lab.astype(jnp.float32), z, z], axis=1)


def _nms_body(pl_ref, out_ref, nx1_r, ny1_r, nx2_r, ny2_r, area_r):
    bx1 = pl_ref[0]
    by1 = pl_ref[1]
    bx2 = pl_ref[2]
    by2 = pl_ref[3]
    s = pl_ref[4]
    labf = pl_ref[5]
    si = jax.lax.bitcast_convert_type(s, jnp.int32)
    row = jax.lax.broadcasted_iota(jnp.int32, (_ROWS, _LANES), 0)
    col = jax.lax.broadcasted_iota(jnp.int32, (_ROWS, _LANES), 1)
    gidx = col * _ROWS + row

    def count(mask):
        return jnp.sum(mask.astype(jnp.int32))

    t = jnp.int32(0)
    for b in range(30, -1, -1):
        tc = t | jnp.int32(1 << b)
        t = jnp.where(count(si >= tc) >= _K, tc, t)
    eqt = si == t
    need = _K - count(si > t)
    x_cut = jnp.int32(0)
    for b in range(14, -1, -1):
        xc = x_cut | jnp.int32(1 << b)
        x_cut = jnp.where(count(eqt & (gidx < xc)) < need, xc, x_cut)
    cand = (si > t) | (eqt & (gidx <= x_cut))

    m4 = jnp.maximum(jnp.maximum(bx1, by1), jnp.maximum(bx2, by2))
    mc = jnp.max(jnp.where(cand, m4, 0.0))
    off = labf * (mc + 1.0)
    nx1 = bx1 + off
    ny1 = by1 + off
    nx2 = bx2 + off
    ny2 = by2 + off
    nx1_r[...] = nx1
    ny1_r[...] = ny1
    nx2_r[...] = nx2
    ny2_r[...] = ny2
    area_r[...] = (nx2 - nx1) * (ny2 - ny1)
    live0 = jnp.where(cand, s, -jnp.inf)

    lanei = jax.lax.broadcasted_iota(jnp.int32, (1, _LANES), 1)

    def step(i, carry):
        live, first = carry
        colmax = jnp.max(live, axis=0, keepdims=True)
        colarg = jnp.argmax(live, axis=0, keepdims=True)
        m = jnp.max(colmax)
        idx = jnp.min(jnp.where(colmax == m,
                                lanei * _ROWS + colarg.astype(jnp.int32),
                                _NPAD))
        idx = jnp.where(m == -jnp.inf, first, idx)
        first = jnp.where(i == 0, idx, first)
        cf = idx // _ROWS
        rf = idx - cf * _ROWS
        cf_v = jnp.full((8, _LANES), cf, jnp.int32)

        def pick(f):
            rowv = pl_ref[f, pl.ds(rf, 1), :]
            g = jnp.take_along_axis(jnp.broadcast_to(rowv, (8, _LANES)),
                                    cf_v, axis=1,
                                    mode="promise_in_bounds")
            return g[0:1, :]

        px1b = pick(0)
        py1b = pick(1)
        px2b = pick(2)
        py2b = pick(3)
        psc = pick(4)
        plab = pick(5)
        poff = plab * (mc + 1.0)
        px1 = px1b + poff
        py1 = py1b + poff
        px2 = px2b + poff
        py2 = py2b + poff
        parea = (px2 - px1) * (py2 - py1)
        ltx = jnp.maximum(px1, nx1_r[...])
        lty = jnp.maximum(py1, ny1_r[...])
        rbx = jnp.minimum(px2, nx2_r[...])
        rby = jnp.minimum(py2, ny2_r[...])
        w = jnp.maximum(rbx - ltx, 0.0)
        h = jnp.maximum(rby - lty, 0.0)
        inter = w * h
        iou = inter / (parea + area_r[...] - inter + 1e-9)
        live = jnp.where((iou > _NMS_THRESH) | (gidx == idx), -jnp.inf, live)
        rowout = jnp.where(lanei == 0, px1b,
                 jnp.where(lanei == 1, py1b,
                 jnp.where(lanei == 2, px2b,
                 jnp.where(lanei == 3, py2b,
                 jnp.where(lanei == 4, psc,
                 jnp.where(lanei == 5, plab, 0.0))))))
        out_ref[pl.ds(i, 1), :] = rowout
        return live, first

    jax.lax.fori_loop(0, _NUM_OUT, step, (live0, jnp.int32(0)))


@jax.jit
def kernel(cls_logits, bbox_regression, bbox_ctrness, anchors):
    grid = _N // _BLK
    packed = pl.pallas_call(
        _score_body,
        grid=(grid,),
        in_specs=[
            pl.BlockSpec((_BLK, _NCLS), lambda i: (i, 0)),
            pl.BlockSpec((_BLK, 1), lambda i: (i, 0)),
            pl.BlockSpec((_BLK, 4), lambda i: (i, 0)),
            pl.BlockSpec((_BLK, 4), lambda i: (i, 0)),
        ],
        out_specs=pl.BlockSpec((_BLK, 8), lambda i: (i, 0)),
        out_shape=jax.ShapeDtypeStruct((_N, 8), jnp.float32),
    )(cls_logits, bbox_ctrness, bbox_regression, anchors)

    padded = jnp.pad(packed, ((0, _NPAD - _N), (0, 0)))
    planes = padded.T.reshape(8, _LANES, _ROWS).transpose(0, 2, 1)

    out = pl.pallas_call(
        _nms_body,
        in_specs=[pl.BlockSpec(memory_space=pltpu.VMEM)],
        out_specs=pl.BlockSpec(memory_space=pltpu.VMEM),
        out_shape=jax.ShapeDtypeStruct((104, _LANES), jnp.float32),
        scratch_shapes=[pltpu.VMEM((_ROWS, _LANES), jnp.float32)] * 5,
    )(planes)
    dets = out[:_NUM_OUT, :5]
    labels_out = out[:_NUM_OUT, 5].astype(jnp.int32)
    return dets, labels_out


# speedup vs baseline: 7.8898x; 1.2153x over previous
import jax
import jax.numpy as jnp
from jax.experimental import pallas as pl
from jax.experimental.pallas import tpu as pltpu

_N = 20000
_ROWS, _LANES = 160, 128
_NPAD = _ROWS * _LANES
_NCLS = 80
_K = 1000
_SCORE_THRESH = 0.2
_NMS_THRESH = 0.6
_NUM_OUT = 100
_IMG = 1024.0


def _fcos_body(lg_ref, aux_ref, out_ref, nx1_r, ny1_r, nx2_r, ny2_r, area_r):
    row = jax.lax.broadcasted_iota(jnp.int32, (_ROWS, _LANES), 0)
    col = jax.lax.broadcasted_iota(jnp.int32, (_ROWS, _LANES), 1)
    gidx = col * _ROWS + row

    maxl = lg_ref[0]
    lab = jnp.zeros((_ROWS, _LANES), jnp.int32)
    for c in range(1, _NCLS):
        x = lg_ref[c]
        upd = x > maxl
        maxl = jnp.where(upd, x, maxl)
        lab = jnp.where(upd, c, lab)
    labf = lab.astype(jnp.float32)
    ctr = aux_ref[0]
    s = jnp.sqrt(jax.nn.sigmoid(maxl) * jax.nn.sigmoid(ctr))
    s = jnp.where(s > _SCORE_THRESH, s, 0.0)
    s = jnp.where(gidx < _N, s, 0.0)

    cx = (aux_ref[5] + aux_ref[7]) / 2.0
    cy = (aux_ref[6] + aux_ref[8]) / 2.0
    bx1 = jnp.clip(cx - aux_ref[1], 0.0, _IMG)
    by1 = jnp.clip(cy - aux_ref[2], 0.0, _IMG)
    bx2 = jnp.clip(cx + aux_ref[3], 0.0, _IMG)
    by2 = jnp.clip(cy + aux_ref[4], 0.0, _IMG)

    si = jax.lax.bitcast_convert_type(s, jnp.int32)

    def count(mask):
        return jnp.sum(mask.astype(jnp.int32))

    t = jnp.int32(0)
    for b in range(30, -1, -1):
        tc = t | jnp.int32(1 << b)
        t = jnp.where(count(si >= tc) >= _K, tc, t)
    eqt = si == t
    need = _K - count(si > t)
    x_cut = jnp.int32(0)
    for b in range(14, -1, -1):
        xc = x_cut | jnp.int32(1 << b)
        x_cut = jnp.where(count(eqt & (gidx < xc)) < need, xc, x_cut)
    cand = (si > t) | (eqt & (gidx <= x_cut))

    m4 = jnp.maximum(jnp.maximum(bx1, by1), jnp.maximum(bx2, by2))
    mc = jnp.max(jnp.where(cand, m4, 0.0))
    off = labf * (mc + 1.0)
    nx1 = bx1 + off
    ny1 = by1 + off
    nx2 = bx2 + off
    ny2 = by2 + off
    nx1_r[...] = nx1
    ny1_r[...] = ny1
    nx2_r[...] = nx2
    ny2_r[...] = ny2
    area_r[...] = (nx2 - nx1) * (ny2 - ny1)
    live0 = jnp.where(cand, s, -jnp.inf)
    lg_ref[0] = bx1
    lg_ref[1] = by1
    lg_ref[2] = bx2
    lg_ref[3] = by2
    lg_ref[4] = jnp.where(cand, s, 0.0)
    lg_ref[5] = labf

    lanei = jax.lax.broadcasted_iota(jnp.int32, (1, _LANES), 1)

    def step(i, carry):
        live, first = carry
        colmax = jnp.max(live, axis=0, keepdims=True)
        colarg = jnp.argmax(live, axis=0, keepdims=True)
        m = jnp.max(colmax)
        idx = jnp.min(jnp.where(colmax == m,
                                lanei * _ROWS + colarg.astype(jnp.int32),
                                _NPAD))
        idx = jnp.where(m == -jnp.inf, first, idx)
        first = jnp.where(i == 0, idx, first)
        cf = idx // _ROWS
        rf = idx - cf * _ROWS
        cf_v = jnp.full((8, _LANES), cf, jnp.int32)

        def pick(f):
            rowv = lg_ref[f, pl.ds(rf, 1), :]
            g = jnp.take_along_axis(jnp.broadcast_to(rowv, (8, _LANES)),
                                    cf_v, axis=1,
                                    mode="promise_in_bounds")
            return g[0:1, :]

        px1b = pick(0)
        py1b = pick(1)
        px2b = pick(2)
        py2b = pick(3)
        psc = pick(4)
        plab = pick(5)
        poff = plab * (mc + 1.0)
        px1 = px1b + poff
        py1 = py1b + poff
        px2 = px2b + poff
        py2 = py2b + poff
        parea = (px2 - px1) * (py2 - py1)
        ltx = jnp.maximum(px1, nx1_r[...])
        lty = jnp.maximum(py1, ny1_r[...])
        rbx = jnp.minimum(px2, nx2_r[...])
        rby = jnp.minimum(py2, ny2_r[...])
        w = jnp.maximum(rbx - ltx, 0.0)
        h = jnp.maximum(rby - lty, 0.0)
        inter = w * h
        iou = inter / (parea + area_r[...] - inter + 1e-9)
        live = jnp.where((iou > _NMS_THRESH) | (gidx == idx), -jnp.inf, live)
        rowout = jnp.where(lanei == 0, px1b,
                 jnp.where(lanei == 1, py1b,
                 jnp.where(lanei == 2, px2b,
                 jnp.where(lanei == 3, py2b,
                 jnp.where(lanei == 4, psc,
                 jnp.where(lanei == 5, plab, 0.0))))))
        out_ref[pl.ds(i, 1), :] = rowout
        return live, first

    jax.lax.fori_loop(0, _NUM_OUT, step, (live0, jnp.int32(0)))


@jax.jit
def kernel(cls_logits, bbox_regression, bbox_ctrness, anchors):
    def fold(x):
        f = x.shape[1]
        xp = jnp.pad(x, ((0, _NPAD - _N), (0, 0)))
        return xp.T.reshape(f, _LANES, _ROWS).transpose(0, 2, 1)

    lg = fold(cls_logits)
    aux = fold(jnp.concatenate(
        [bbox_ctrness, bbox_regression, anchors], axis=1))

    out = pl.pallas_call(
        _fcos_body,
        in_specs=[pl.BlockSpec(memory_space=pltpu.VMEM)] * 2,
        out_specs=pl.BlockSpec(memory_space=pltpu.VMEM),
        out_shape=jax.ShapeDtypeStruct((104, _LANES), jnp.float32),
        scratch_shapes=[pltpu.VMEM((_ROWS, _LANES), jnp.float32)] * 5,
    )(lg, aux)
    dets = out[:_NUM_OUT, :5]
    labels_out = out[:_NUM_OUT, 5].astype(jnp.int32)
    return dets, labels_out
